# Initial kernel scaffold; baseline (speedup 1.0000x reference)
import jax
import jax.numpy as jnp
from jax.experimental import pallas as pl


def kernel(x, W, delta):
    raise NotImplementedError("write your pallas kernel here")


# speedup vs baseline: 3.7624x; 3.7624x over previous
import functools

import jax
import jax.numpy as jnp
from jax import lax
from jax.experimental import pallas as pl
from jax.experimental.pallas import tpu as pltpu
from jax.experimental.pallas import tpu_sc as plsc

NUM_EMB = 100000
DIM = 64
BAND_LO = 32000
N_TRAIN = 16
BATCH = 4096
HIST = 200
TOTAL = BATCH * HIST
IDX_COLS = 128
IDX_ROWS = TOTAL // IDX_COLS
CHUNK_ROWS = 4
CHUNK = CHUNK_ROWS * IDX_COLS
LANES = 16


def _sc_body(x_hbm, w_hbm, delta_hbm, out_hbm, idx_v, rows_v, dflat_v, sem):
    info = plsc.get_sparse_core_info()
    nc = info.num_cores
    nw = nc * info.num_subcores
    rows_per_w = IDX_ROWS // nw
    n_chunks = rows_per_w // CHUNK_ROWS

    wid = lax.axis_index("s") * nc + lax.axis_index("c")
    row_base = wid * rows_per_w

    pltpu.sync_copy(delta_hbm, dflat_v)

    lane = lax.iota(jnp.int32, LANES)

    def chunk_body(c, _):
        r0 = row_base + c * CHUNK_ROWS
        pltpu.sync_copy(x_hbm.at[pl.ds(r0, CHUNK_ROWS)], idx_v)

        cps = [
            pltpu.async_copy(
                w_hbm.at[idx_v.at[j]],
                rows_v.at[pl.ds(j * IDX_COLS, IDX_COLS)],
                sem,
            )
            for j in range(CHUNK_ROWS)
        ]
        for cp in cps:
            cp.wait()

        def fix_slice(s, _):
            idxs = idx_v[s // (IDX_COLS // LANES),
                         pl.ds((s % (IDX_COLS // LANES)) * LANES, LANES)]
            rel = idxs - BAND_LO
            m = (rel >= 0) & (rel < N_TRAIN)

            @pl.when(jnp.any(m))
            def _():
                relc = jnp.where(m, rel, 0)
                rowid = s * LANES + lane
                for col in range(DIM):
                    dvals = plsc.load_gather(
                        dflat_v, [col * N_TRAIN + relc], mask=m)
                    plsc.addupdate_scatter(
                        rows_v,
                        [rowid, jnp.full((LANES,), col, jnp.int32)],
                        dvals,
                        mask=m,
                    )
            return _

        lax.fori_loop(0, CHUNK // LANES, fix_slice, None, unroll=False)

        pltpu.sync_copy(rows_v, out_hbm.at[pl.ds(r0 * IDX_COLS, CHUNK)])
        return _

    lax.fori_loop(0, n_chunks, chunk_body, None, unroll=False)


def kernel(x, W, delta):
    x2d = x.astype(jnp.int32).reshape(IDX_ROWS, IDX_COLS)
    mesh = plsc.VectorSubcoreMesh(core_axis_name="c", subcore_axis_name="s")
    run = pl.kernel(
        _sc_body,
        out_type=jax.ShapeDtypeStruct((TOTAL, DIM), jnp.float32),
        mesh=mesh,
        scratch_types=[
            pltpu.VMEM((CHUNK_ROWS, IDX_COLS), jnp.int32),
            pltpu.VMEM((CHUNK, DIM), jnp.float32),
            pltpu.VMEM((N_TRAIN * DIM,), jnp.float32),
            pltpu.SemaphoreType.DMA,
        ],
        compiler_params=pltpu.CompilerParams(
            needs_layout_passes=False, use_tc_tiling_on_sc=False),
    )
    out = run(x2d, W, delta)
    return out.reshape(BATCH, HIST, DIM)


# speedup vs baseline: 4.2772x; 1.1368x over previous
import jax
import jax.numpy as jnp
from jax import lax
from jax.experimental import pallas as pl
from jax.experimental.pallas import tpu as pltpu
from jax.experimental.pallas import tpu_sc as plsc

NUM_EMB = 100000
DIM = 64
BAND_LO = 32000
N_TRAIN = 16
BATCH = 4096
HIST = 200
TOTAL = BATCH * HIST
IDX_COLS = 128
IDX_ROWS = TOTAL // IDX_COLS
CHUNK_ROWS = 4
CHUNK = CHUNK_ROWS * IDX_COLS
LANES = 16


def _sc_body(x_hbm, w_hbm, delta_hbm, out_hbm,
             idx0, idx1, rows0, rows1, dflat_v,
             gsem0, gsem1, wsem, isem):
    info = plsc.get_sparse_core_info()
    nc = info.num_cores
    nw = nc * info.num_subcores
    rows_per_w = IDX_ROWS // nw
    n_chunks = rows_per_w // CHUNK_ROWS

    idxb = (idx0, idx1)
    rowsb = (rows0, rows1)
    gsem = (gsem0, gsem1)

    wid = lax.axis_index("s") * nc + lax.axis_index("c")
    row_base = wid * rows_per_w

    pltpu.sync_copy(delta_hbm, dflat_v)

    lane = lax.iota(jnp.int32, LANES)

    def issue_idx(c, p):
        pltpu.async_copy(
            x_hbm.at[pl.ds(row_base + c * CHUNK_ROWS, CHUNK_ROWS)],
            idxb[p], isem)

    def drain_idx(p):
        pltpu.make_async_copy(
            x_hbm.at[pl.ds(0, CHUNK_ROWS)], idxb[p], isem).wait()

    def issue_gathers(p):
        for j in range(CHUNK_ROWS):
            pltpu.async_copy(
                w_hbm.at[idxb[p].at[j]],
                rowsb[p].at[pl.ds(j * IDX_COLS, IDX_COLS)],
                gsem[p])

    def drain_gathers(p):
        for j in range(CHUNK_ROWS):
            pltpu.make_async_copy(
                w_hbm.at[idxb[p].at[j]],
                rowsb[p].at[pl.ds(j * IDX_COLS, IDX_COLS)],
                gsem[p]).wait()

    def issue_write(c, p):
        pltpu.async_copy(
            rowsb[p],
            out_hbm.at[pl.ds((row_base + c * CHUNK_ROWS) * IDX_COLS, CHUNK)],
            wsem)

    def drain_write(p):
        pltpu.make_async_copy(
            rowsb[p], out_hbm.at[pl.ds(0, CHUNK)], wsem).wait()

    def fixup(p):
        def fix_slice(s, _):
            idxs = idxb[p][s // (IDX_COLS // LANES),
                           pl.ds((s % (IDX_COLS // LANES)) * LANES, LANES)]
            rel = idxs - BAND_LO
            m = (rel >= 0) & (rel < N_TRAIN)

            @pl.when(jnp.any(m))
            def _():
                relc = jnp.where(m, rel, 0)
                rowid = s * LANES + lane
                for col in range(DIM):
                    dvals = plsc.load_gather(
                        dflat_v, [col * N_TRAIN + relc], mask=m)
                    plsc.addupdate_scatter(
                        rowsb[p],
                        [rowid, jnp.full((LANES,), col, jnp.int32)],
                        dvals,
                        mask=m,
                    )
            return _

        lax.fori_loop(0, CHUNK // LANES, fix_slice, None, unroll=False)

    pltpu.sync_copy(x_hbm.at[pl.ds(row_base, CHUNK_ROWS)], idx0)
    issue_gathers(0)
    issue_idx(1, 1)

    def pair_body(g, _):
        for p in range(2):
            c = g * 2 + p
            q = 1 - p

            @pl.when(c < n_chunks - 1)
            def _():
                drain_idx(q)

            @pl.when(c >= 1)
            def _():
                drain_write(q)

            @pl.when(c < n_chunks - 1)
            def _():
                issue_gathers(q)

            drain_gathers(p)
            fixup(p)

            @pl.when(c < n_chunks - 2)
            def _():
                issue_idx(c + 2, p)

            issue_write(c, p)
        return _

    lax.fori_loop(0, n_chunks // 2, pair_body, None, unroll=False)
    drain_write((n_chunks - 1) % 2)


def kernel(x, W, delta):
    x2d = x.astype(jnp.int32).reshape(IDX_ROWS, IDX_COLS)
    mesh = plsc.VectorSubcoreMesh(core_axis_name="c", subcore_axis_name="s")
    run = pl.kernel(
        _sc_body,
        out_type=jax.ShapeDtypeStruct((TOTAL, DIM), jnp.float32),
        mesh=mesh,
        scratch_types=[
            pltpu.VMEM((CHUNK_ROWS, IDX_COLS), jnp.int32),
            pltpu.VMEM((CHUNK_ROWS, IDX_COLS), jnp.int32),
            pltpu.VMEM((CHUNK, DIM), jnp.float32),
            pltpu.VMEM((CHUNK, DIM), jnp.float32),
            pltpu.VMEM((N_TRAIN * DIM,), jnp.float32),
            pltpu.SemaphoreType.DMA,
            pltpu.SemaphoreType.DMA,
            pltpu.SemaphoreType.DMA,
            pltpu.SemaphoreType.DMA,
        ],
        compiler_params=pltpu.CompilerParams(
            needs_layout_passes=False, use_tc_tiling_on_sc=False),
    )
    out = run(x2d, W, delta)
    return out.reshape(BATCH, HIST, DIM)
